# baseline (speedup 1.0000x reference)
import math
import functools

import jax
import jax.numpy as jnp
from jax import lax
from jax.experimental import pallas as pl
from jax.experimental.pallas import tpu as pltpu
from jax.experimental.pallas import tpu_sc as plsc

D_MODEL = 768
N_EXPERTS = 8
D_FF = 4 * D_MODEL
VOCAB = 256
T_SEQ = 2048
TOL = 0.15


T_BLK = 512
_PI = math.pi
_HALF_PI = math.pi / 2
_SIN_C = (0.9999999998895754, -0.16666666541211608, 0.008333329259275685,
          -0.0001984070238109481, 2.7518836060965494e-06,
          -2.3794424157310798e-08)
_COS_C = (0.9999999999992444, -0.499999999970178, 0.04166666647310349,
          -0.001388888417524428, 2.4801040276741908e-05,
          -2.752468296544026e-07, 1.990767507756418e-09)


def _sc_gather_body(idx_hbm, table_hbm, out_hbm, idx_v, rows_v, sem):
    wid = lax.axis_index("s") * 2 + lax.axis_index("c")
    base = wid * (T_SEQ // 32)
    pltpu.sync_copy(idx_hbm.at[pl.ds(base, T_SEQ // 32)], idx_v)
    pltpu.async_copy(table_hbm.at[idx_v], rows_v, sem).wait()
    pltpu.sync_copy(rows_v, out_hbm.at[pl.ds(base, T_SEQ // 32)])


def _pre_body(x_ref, phiw_ref, phib_ref, ampw_ref, ampb_ref,
              rw_ref, rb_ref,
              ybf_ref, gates_ref, state_ref,
              a_ref, s_ref, g_ref, st_ref, y_ref):
    i = pl.program_id(0)
    x = x_ref[:, :]

    raw = jnp.dot(x, phiw_ref[:, :],
                  preferred_element_type=jnp.float32) + phib_ref[:, :]
    ang = jnp.tanh(raw) * math.pi
    step = math.pi / 9.0
    v = ang * (1.0 / step)
    harm = jnp.floor(v + 0.5) * step
    ang = jnp.where(jnp.abs(ang - harm) < TOL, harm, ang)
    absa = jnp.abs(ang)
    red = jnp.where(absa > _HALF_PI, jnp.sign(ang) * _PI - ang, ang)
    z = red * red
    sp = _SIN_C[5]
    for c in (_SIN_C[4], _SIN_C[3], _SIN_C[2], _SIN_C[1], _SIN_C[0]):
        sp = sp * z + c
    sn = sp * red
    cp = _COS_C[6]
    for c in (_COS_C[5], _COS_C[4], _COS_C[3], _COS_C[2], _COS_C[1],
              _COS_C[0]):
        cp = cp * z + c
    cs = jnp.where(absa > _HALF_PI, -cp, cp)
    a_ref[:, :] = cs + sn
    s_ref[:, :] = sn
    g_ref[:, :] = jax.nn.sigmoid(
        jnp.dot(x, ampw_ref[:, :], preferred_element_type=jnp.float32) + ampb_ref[:, :])

    @pl.when(i == 0)
    def _init_state():
        st_ref[:, :] = jnp.zeros((1, D_MODEL), jnp.float32)

    def body(t, st):
        new = jnp.clip(a_ref[pl.ds(t, 1), :] * st - s_ref[pl.ds(t, 1), :],
                       -1.0, 1.0)
        y_ref[pl.ds(t, 1), :] = g_ref[pl.ds(t, 1), :] * new
        return new

    st_ref[:, :] = lax.fori_loop(0, T_BLK, body, st_ref[:, :])

    @pl.when(i == pl.num_programs(0) - 1)
    def _emit_state():
        state_ref[:, :] = st_ref[:, :]

    y = y_ref[:, :]
    ybf_ref[:, :] = y.astype(jnp.bfloat16)

    lg = jnp.dot(y, rw_ref[:, :],
                 preferred_element_type=jnp.float32) + rb_ref[:, :]
    m1 = jnp.max(lg, axis=1, keepdims=True)
    masked = jnp.where(lg >= m1, -jnp.inf, lg)
    m2 = jnp.max(masked, axis=1, keepdims=True)
    g1 = 1.0 / (1.0 + jnp.exp(m2 - m1))
    gates_ref[:, :] = jnp.where(lg >= m1, g1,
                                jnp.where(lg >= m2, 1.0 - g1, 0.0))


FF_BLK = 3072


def _moe_body(ybf_ref, gates_ref, w1_ref, b1_ref, w2_ref, b2_ref,
              hw_ref, hb_ref, logits_ref, acc_ref):
    e = pl.program_id(0)
    f = pl.program_id(1)

    @pl.when(jnp.logical_and(e == 0, f == 0))
    def _init():
        acc_ref[:, :] = jnp.zeros_like(acc_ref)

    xb = ybf_ref[:, :]
    sub = 768
    n_sub = FF_BLK // sub
    part = None
    for u in range(n_sub):
        w1b = w1_ref[0, :, u * sub:(u + 1) * sub].astype(jnp.bfloat16)
        h = jnp.dot(xb, w1b, preferred_element_type=jnp.float32)
        h = (h + b1_ref[0, :, u * sub:(u + 1) * sub]).astype(jnp.bfloat16)
        h = jax.nn.gelu(h)
        w2b = w2_ref[0, u * sub:(u + 1) * sub, :].astype(jnp.bfloat16)
        p = jnp.dot(h, w2b, preferred_element_type=jnp.float32)
        part = p if part is None else part + p

    lane = lax.broadcasted_iota(jnp.int32, (T_SEQ, N_EXPERTS), 1)
    gate = jnp.sum(jnp.where(lane == e, gates_ref[:, :], 0.0), axis=1,
                   keepdims=True)

    @pl.when(f == 0)
    def _bias():
        acc_ref[:, :] += gate * (part + b2_ref[0, :, :])

    @pl.when(f != 0)
    def _nobias():
        acc_ref[:, :] += gate * part

    @pl.when(jnp.logical_and(e == pl.num_programs(0) - 1,
                             f == pl.num_programs(1) - 1))
    def _head():
        ob = acc_ref[:, :].astype(jnp.bfloat16)
        logits_ref[:, :] = (jnp.dot(ob, hw_ref[:, :],
                                    preferred_element_type=jnp.float32)
                            + hb_ref[:, :])


def kernel(byte_seq, emb, phi_w, phi_b, amp_w, amp_b, router_w, router_b,
           w1, b1, w2, b2, head_w, head_b):
    f32 = jnp.float32

    x = pl.kernel(
        _sc_gather_body,
        mesh=plsc.VectorSubcoreMesh(core_axis_name="c", subcore_axis_name="s"),
        out_type=jax.ShapeDtypeStruct((T_SEQ, D_MODEL), f32),
        scratch_types=[
            pltpu.VMEM((T_SEQ // 32,), jnp.int32),
            pltpu.VMEM((T_SEQ // 32, D_MODEL), f32),
            pltpu.SemaphoreType.DMA,
        ],
    )(byte_seq.reshape(T_SEQ).astype(jnp.int32), emb)

    n_tb = T_SEQ // T_BLK
    ybf, gates, state = pl.pallas_call(
        _pre_body,
        grid=(n_tb,),
        in_specs=[
            pl.BlockSpec((T_BLK, D_MODEL), lambda i: (i, 0)),
            pl.BlockSpec((D_MODEL, D_MODEL), lambda i: (0, 0)),
            pl.BlockSpec((1, D_MODEL), lambda i: (0, 0)),
            pl.BlockSpec((D_MODEL, D_MODEL), lambda i: (0, 0)),
            pl.BlockSpec((1, D_MODEL), lambda i: (0, 0)),
            pl.BlockSpec((D_MODEL, N_EXPERTS), lambda i: (0, 0)),
            pl.BlockSpec((1, N_EXPERTS), lambda i: (0, 0)),
        ],
        out_specs=[
            pl.BlockSpec((T_BLK, D_MODEL), lambda i: (i, 0)),
            pl.BlockSpec((T_BLK, N_EXPERTS), lambda i: (i, 0)),
            pl.BlockSpec((1, D_MODEL), lambda i: (0, 0)),
        ],
        out_shape=[
            jax.ShapeDtypeStruct((T_SEQ, D_MODEL), jnp.bfloat16),
            jax.ShapeDtypeStruct((T_SEQ, N_EXPERTS), f32),
            jax.ShapeDtypeStruct((1, D_MODEL), f32),
        ],
        scratch_shapes=[
            pltpu.VMEM((T_BLK, D_MODEL), f32),
            pltpu.VMEM((T_BLK, D_MODEL), f32),
            pltpu.VMEM((T_BLK, D_MODEL), f32),
            pltpu.VMEM((1, D_MODEL), f32),
            pltpu.VMEM((T_BLK, D_MODEL), f32),
        ],
        compiler_params=pltpu.CompilerParams(
            dimension_semantics=("arbitrary",),
            vmem_limit_bytes=100 * 2**20,
        ),
    )(x, phi_w, phi_b.reshape(1, D_MODEL),
      amp_w, amp_b.reshape(1, D_MODEL),
      router_w, router_b.reshape(1, N_EXPERTS))

    ff_blk = FF_BLK
    n_ff = D_FF // ff_blk
    logits = pl.pallas_call(
        _moe_body,
        grid=(N_EXPERTS, n_ff),
        in_specs=[
            pl.BlockSpec((T_SEQ, D_MODEL), lambda e, f: (0, 0)),
            pl.BlockSpec((T_SEQ, N_EXPERTS), lambda e, f: (0, 0)),
            pl.BlockSpec((1, D_MODEL, ff_blk), lambda e, f: (e, 0, f)),
            pl.BlockSpec((1, 1, ff_blk), lambda e, f: (e, 0, f)),
            pl.BlockSpec((1, ff_blk, D_MODEL), lambda e, f: (e, f, 0)),
            pl.BlockSpec((1, 1, D_MODEL), lambda e, f: (e, 0, 0)),
            pl.BlockSpec((D_MODEL, VOCAB), lambda e, f: (0, 0)),
            pl.BlockSpec((1, VOCAB), lambda e, f: (0, 0)),
        ],
        out_specs=pl.BlockSpec((T_SEQ, VOCAB), lambda e, f: (0, 0)),
        out_shape=jax.ShapeDtypeStruct((T_SEQ, VOCAB), f32),
        scratch_shapes=[pltpu.VMEM((T_SEQ, D_MODEL), f32)],
        compiler_params=pltpu.CompilerParams(
            dimension_semantics=("arbitrary", "arbitrary"),
            vmem_limit_bytes=100 * 2**20,
        ),
    )(ybf, gates, w1, b1.reshape(N_EXPERTS, 1, D_FF), w2,
      b2.reshape(N_EXPERTS, 1, D_MODEL),
      head_w.astype(jnp.bfloat16), head_b.reshape(1, VOCAB))

    entropy_loss = jnp.zeros((), f32)
    return logits.reshape(1, T_SEQ, VOCAB), state, entropy_loss


# speedup vs baseline: 1.0009x; 1.0009x over previous
import math

import jax
import jax.numpy as jnp
from jax import lax
from jax.experimental import pallas as pl
from jax.experimental.pallas import tpu as pltpu
from jax.experimental.pallas import tpu_sc as plsc

D_MODEL = 768
N_EXPERTS = 8
D_FF = 4 * D_MODEL
VOCAB = 256
T_SEQ = 2048
TOL = 0.15


T_BLK = 512
_PI = math.pi
_HALF_PI = math.pi / 2
_SIN_C = (0.9999999998895754, -0.16666666541211608, 0.008333329259275685,
          -0.0001984070238109481, 2.7518836060965494e-06,
          -2.3794424157310798e-08)
_COS_C = (0.9999999999992444, -0.499999999970178, 0.04166666647310349,
          -0.001388888417524428, 2.4801040276741908e-05,
          -2.752468296544026e-07, 1.990767507756418e-09)


def _sc_gather_body(idx_hbm, table_hbm, out_hbm, idx_v, rows_v, sem):
    wid = lax.axis_index("s") * 2 + lax.axis_index("c")
    base = wid * (T_SEQ // 32)
    pltpu.sync_copy(idx_hbm.at[pl.ds(base, T_SEQ // 32)], idx_v)
    pltpu.async_copy(table_hbm.at[idx_v], rows_v, sem).wait()
    pltpu.sync_copy(rows_v, out_hbm.at[pl.ds(base, T_SEQ // 32)])


def _pre_body(x_ref, phiw_ref, phib_ref, ampw_ref, ampb_ref,
              rw_ref, rb_ref,
              ybf_ref, gates_ref, state_ref,
              a_ref, s_ref, g_ref, st_ref, y_ref):
    i = pl.program_id(0)
    x = x_ref[:, :]

    raw = jnp.dot(x, phiw_ref[:, :],
                  preferred_element_type=jnp.float32) + phib_ref[:, :]
    ang = jnp.tanh(raw) * math.pi
    step = math.pi / 9.0
    v = ang * (1.0 / step)
    harm = jnp.floor(v + 0.5) * step
    ang = jnp.where(jnp.abs(ang - harm) < TOL, harm, ang)
    absa = jnp.abs(ang)
    red = jnp.where(absa > _HALF_PI, jnp.sign(ang) * _PI - ang, ang)
    z = red * red
    sp = _SIN_C[5]
    for c in (_SIN_C[4], _SIN_C[3], _SIN_C[2], _SIN_C[1], _SIN_C[0]):
        sp = sp * z + c
    sn = sp * red
    cp = _COS_C[6]
    for c in (_COS_C[5], _COS_C[4], _COS_C[3], _COS_C[2], _COS_C[1],
              _COS_C[0]):
        cp = cp * z + c
    cs = jnp.where(absa > _HALF_PI, -cp, cp)
    a_ref[:, :] = cs + sn
    s_ref[:, :] = sn
    g_ref[:, :] = jax.nn.sigmoid(
        jnp.dot(x, ampw_ref[:, :], preferred_element_type=jnp.float32) + ampb_ref[:, :])

    @pl.when(i == 0)
    def _init_state():
        st_ref[:, :] = jnp.zeros((1, D_MODEL), jnp.float32)

    def body(t, st):
        new = jnp.clip(a_ref[pl.ds(t, 1), :] * st - s_ref[pl.ds(t, 1), :],
                       -1.0, 1.0)
        y_ref[pl.ds(t, 1), :] = g_ref[pl.ds(t, 1), :] * new
        return new

    st_ref[:, :] = lax.fori_loop(0, T_BLK, body, st_ref[:, :])

    @pl.when(i == pl.num_programs(0) - 1)
    def _emit_state():
        state_ref[:, :] = st_ref[:, :]

    y = y_ref[:, :]
    ybf_ref[:, :] = y.astype(jnp.bfloat16)

    lg = jnp.dot(y, rw_ref[:, :],
                 preferred_element_type=jnp.float32) + rb_ref[:, :]
    m1 = jnp.max(lg, axis=1, keepdims=True)
    masked = jnp.where(lg >= m1, -jnp.inf, lg)
    m2 = jnp.max(masked, axis=1, keepdims=True)
    g1 = 1.0 / (1.0 + jnp.exp(m2 - m1))
    gates_ref[:, :] = jnp.where(lg >= m1, g1,
                                jnp.where(lg >= m2, 1.0 - g1, 0.0))


FF_BLK = 3072


def _moe_body(ybf_ref, gates_ref, w1_ref, b1_ref, w2_ref, b2_ref,
              hw_ref, hb_ref, logits_ref, acc_ref):
    e = pl.program_id(0)
    f = pl.program_id(1)

    @pl.when(jnp.logical_and(e == 0, f == 0))
    def _init():
        acc_ref[:, :] = jnp.zeros_like(acc_ref)

    xb = ybf_ref[:, :]
    sub = 768
    n_sub = FF_BLK // sub
    part = None
    for u in range(n_sub):
        w1b = w1_ref[0, :, u * sub:(u + 1) * sub].astype(jnp.bfloat16)
        h = jnp.dot(xb, w1b, preferred_element_type=jnp.float32)
        h = (h + b1_ref[0, :, u * sub:(u + 1) * sub]).astype(jnp.bfloat16)
        h = jax.nn.gelu(h)
        w2b = w2_ref[0, u * sub:(u + 1) * sub, :].astype(jnp.bfloat16)
        p = jnp.dot(h, w2b, preferred_element_type=jnp.float32)
        part = p if part is None else part + p

    lane = lax.broadcasted_iota(jnp.int32, (T_SEQ, N_EXPERTS), 1)
    gate = jnp.sum(jnp.where(lane == e, gates_ref[:, :], 0.0), axis=1,
                   keepdims=True)

    @pl.when(f == 0)
    def _bias():
        acc_ref[:, :] += gate * (part + b2_ref[0, :, :])

    @pl.when(f != 0)
    def _nobias():
        acc_ref[:, :] += gate * part

    @pl.when(jnp.logical_and(e == pl.num_programs(0) - 1,
                             f == pl.num_programs(1) - 1))
    def _head():
        ob = acc_ref[:, :].astype(jnp.bfloat16)
        logits_ref[:, :] = (jnp.dot(ob, hw_ref[:, :],
                                    preferred_element_type=jnp.float32)
                            + hb_ref[:, :])


def kernel(byte_seq, emb, phi_w, phi_b, amp_w, amp_b, router_w, router_b,
           w1, b1, w2, b2, head_w, head_b):
    f32 = jnp.float32

    x = pl.kernel(
        _sc_gather_body,
        mesh=plsc.VectorSubcoreMesh(core_axis_name="c", subcore_axis_name="s"),
        out_type=jax.ShapeDtypeStruct((T_SEQ, D_MODEL), f32),
        scratch_types=[
            pltpu.VMEM((T_SEQ // 32,), jnp.int32),
            pltpu.VMEM((T_SEQ // 32, D_MODEL), f32),
            pltpu.SemaphoreType.DMA,
        ],
    )(byte_seq.reshape(T_SEQ).astype(jnp.int32), emb)

    n_tb = T_SEQ // T_BLK
    ybf, gates, state = pl.pallas_call(
        _pre_body,
        grid=(n_tb,),
        in_specs=[
            pl.BlockSpec((T_BLK, D_MODEL), lambda i: (i, 0)),
            pl.BlockSpec((D_MODEL, D_MODEL), lambda i: (0, 0)),
            pl.BlockSpec((1, D_MODEL), lambda i: (0, 0)),
            pl.BlockSpec((D_MODEL, D_MODEL), lambda i: (0, 0)),
            pl.BlockSpec((1, D_MODEL), lambda i: (0, 0)),
            pl.BlockSpec((D_MODEL, N_EXPERTS), lambda i: (0, 0)),
            pl.BlockSpec((1, N_EXPERTS), lambda i: (0, 0)),
        ],
        out_specs=[
            pl.BlockSpec((T_BLK, D_MODEL), lambda i: (i, 0)),
            pl.BlockSpec((T_BLK, N_EXPERTS), lambda i: (i, 0)),
            pl.BlockSpec((1, D_MODEL), lambda i: (0, 0)),
        ],
        out_shape=[
            jax.ShapeDtypeStruct((T_SEQ, D_MODEL), jnp.bfloat16),
            jax.ShapeDtypeStruct((T_SEQ, N_EXPERTS), f32),
            jax.ShapeDtypeStruct((1, D_MODEL), f32),
        ],
        scratch_shapes=[
            pltpu.VMEM((T_BLK, D_MODEL), f32),
            pltpu.VMEM((T_BLK, D_MODEL), f32),
            pltpu.VMEM((T_BLK, D_MODEL), f32),
            pltpu.VMEM((1, D_MODEL), f32),
            pltpu.VMEM((T_BLK, D_MODEL), f32),
        ],
        compiler_params=pltpu.CompilerParams(
            dimension_semantics=("arbitrary",),
            vmem_limit_bytes=100 * 2**20,
        ),
    )(x, phi_w, phi_b.reshape(1, D_MODEL),
      amp_w, amp_b.reshape(1, D_MODEL),
      router_w, router_b.reshape(1, N_EXPERTS))

    ff_blk = FF_BLK
    n_ff = D_FF // ff_blk
    logits = pl.pallas_call(
        _moe_body,
        grid=(N_EXPERTS, n_ff),
        in_specs=[
            pl.BlockSpec((T_SEQ, D_MODEL), lambda e, f: (0, 0)),
            pl.BlockSpec((T_SEQ, N_EXPERTS), lambda e, f: (0, 0)),
            pl.BlockSpec((1, D_MODEL, ff_blk), lambda e, f: (e, 0, f)),
            pl.BlockSpec((1, 1, ff_blk), lambda e, f: (e, 0, f)),
            pl.BlockSpec((1, ff_blk, D_MODEL), lambda e, f: (e, f, 0)),
            pl.BlockSpec((1, 1, D_MODEL), lambda e, f: (e, 0, 0)),
            pl.BlockSpec((D_MODEL, VOCAB), lambda e, f: (0, 0)),
            pl.BlockSpec((1, VOCAB), lambda e, f: (0, 0)),
        ],
        out_specs=pl.BlockSpec((T_SEQ, VOCAB), lambda e, f: (0, 0)),
        out_shape=jax.ShapeDtypeStruct((T_SEQ, VOCAB), f32),
        scratch_shapes=[pltpu.VMEM((T_SEQ, D_MODEL), f32)],
        compiler_params=pltpu.CompilerParams(
            dimension_semantics=("arbitrary", "arbitrary"),
            vmem_limit_bytes=100 * 2**20,
        ),
    )(ybf, gates, w1, b1.reshape(N_EXPERTS, 1, D_FF), w2,
      b2.reshape(N_EXPERTS, 1, D_MODEL),
      head_w.astype(jnp.bfloat16), head_b.reshape(1, VOCAB))

    entropy_loss = jnp.zeros((), f32)
    return logits.reshape(1, T_SEQ, VOCAB), state, entropy_loss
